# Initial kernel scaffold; baseline (speedup 1.0000x reference)
import jax
import jax.numpy as jnp
from jax.experimental import pallas as pl


def kernel(x):
    raise NotImplementedError("write your pallas kernel here")


# speedup vs baseline: 3.2288x; 3.2288x over previous
import functools

import jax
import jax.numpy as jnp
from jax import lax
from jax.experimental import pallas as pl
from jax.experimental.pallas import tpu as pltpu
from jax.experimental.pallas import tpu_sc as plsc

B, C, H, W = 4, 3, 384, 384
HW = H * W
NCORES, NSUB, L = 2, 16, 16
ROWS_PER_TILE = H // NSUB
TILE_ELEMS = ROWS_PER_TILE * W
HALO = 8
WIN = ROWS_PER_TILE + 2 * HALO
FB_CHUNK = 32
GROUPS = W // L
BIG = 0x7FFFFFFF
MAGIC = 12582912.0


def _rne(x):
    return (x + MAGIC) - MAGIC


def _take16(x, idx):
    return lax.gather(
        x, idx[:, None],
        lax.GatherDimensionNumbers(
            offset_dims=(), collapsed_slice_dims=(0,), start_index_map=(0,)),
        slice_sizes=(1,),
        mode=lax.GatherScatterMode.PROMISE_IN_BOUNDS)


def _sc_warp_body(x_hbm, out_hbm, fxw, fyw, imw, vr, vg, vb, flg_v, flg_all, shared):
    core = lax.axis_index("c")
    sub = lax.axis_index("s")
    row0 = sub * ROWS_PER_TILE
    lane = lax.iota(jnp.int32, L)
    lane_f = lane.astype(jnp.float32)
    nxt = jnp.minimum(lane + 1, L - 1)
    zeros16 = jnp.zeros((L,), jnp.float32)

    def scatter_group(wr, g, base_row, track_outliers, ofl):
        r_glob = base_row + wr
        r_f = r_glob.astype(jnp.float32)
        c0 = g * L
        fx = fxw[wr, pl.ds(c0, L)]
        fy = fyw[wr, pl.ds(c0, L)]
        jj = c0.astype(jnp.float32) + lane_f
        drf = _rne(fy + r_f)
        dcf = _rne(fx + jj)
        vglob = (drf >= 0.0) & (drf < float(H)) & (dcf >= 0.0) & (dcf < float(W))
        dr = jnp.clip(drf, 0.0, float(H - 1)).astype(jnp.int32)
        dc = jnp.clip(dcf, 0.0, float(W - 1)).astype(jnp.int32)
        mine = vglob & (dr >= row0) & (dr < row0 + ROWS_PER_TILE)
        off = jnp.where(mine, (dr - row0) * W + dc, -1 - lane)
        dup_later = jnp.zeros((L,), jnp.bool_)
        for d in range(1, L):
            offd = _take16(off, jnp.minimum(lane + d, L - 1))
            dup_later = dup_later | ((offd == off) & (lane + d < L))
        keep = mine & jnp.logical_not(dup_later)
        rv = imw[0, wr, pl.ds(c0, L)]
        gv = imw[1, wr, pl.ds(c0, L)]
        bv = imw[2, wr, pl.ds(c0, L)]
        plsc.store_scatter(vr, [off], rv, mask=keep)
        plsc.store_scatter(vg, [off], gv, mask=keep)
        plsc.store_scatter(vb, [off], bv, mask=keep)
        if track_outliers:
            own = (r_glob >= row0) & (r_glob < row0 + ROWS_PER_TILE)
            far = vglob & (jnp.abs(drf - r_f) > float(HALO))
            ofl = ofl | jnp.where(own, far.astype(jnp.int32), 0)
        return ofl

    def zero_vals():
        def zbody(i, _):
            vr[pl.ds(i * L, L)] = zeros16
            vg[pl.ds(i * L, L)] = zeros16
            vb[pl.ds(i * L, L)] = zeros16
            return 0
        lax.fori_loop(0, TILE_ELEMS // L, zbody, 0)

    def load_window(b, r_start, nrows):
        r_start = pl.multiple_of(r_start, 8)
        pltpu.sync_copy(x_hbm.at[b, 6, pl.ds(r_start, nrows)], fxw.at[pl.ds(0, nrows)])
        pltpu.sync_copy(x_hbm.at[b, 7, pl.ds(r_start, nrows)], fyw.at[pl.ds(0, nrows)])
        for ch in range(3):
            pltpu.sync_copy(x_hbm.at[b, ch, pl.ds(r_start, nrows)],
                            imw.at[ch, pl.ds(0, nrows)])

    for k in range(2):
        b = core * 2 + k
        zero_vals()

        w0 = jnp.clip(row0 - HALO, 0, H - WIN)
        load_window(b, w0, WIN)

        def row_body(wr, ofl):
            def g_body(g, ofl_in):
                return scatter_group(wr, g, w0, True, ofl_in)
            return lax.fori_loop(0, GROUPS, g_body, ofl)

        ofl = lax.fori_loop(0, WIN, row_body, jnp.zeros((L,), jnp.int32))

        flg_v[pl.ds(0, L)] = ofl
        pltpu.sync_copy(flg_v, shared.at[k, sub])
        plsc.subcore_barrier()
        pltpu.sync_copy(shared.at[k], flg_all)
        def or_body(i, a):
            return a | flg_all[i, pl.ds(0, L)]
        acc = lax.fori_loop(0, NSUB, or_body, jnp.zeros((L,), jnp.int32))
        any_out = jnp.any(acc != 0)

        @pl.when(any_out)
        def _fallback():
            zero_vals()
            def chunk_body(ci, _):
                r_start = ci * FB_CHUNK
                load_window(b, r_start, FB_CHUNK)
                def row_body_fb(wr, __):
                    def g_body_fb(g, ___):
                        scatter_group(wr, g, r_start, False, None)
                        return 0
                    return lax.fori_loop(0, GROUPS, g_body_fb, 0)
                lax.fori_loop(0, FB_CHUNK, row_body_fb, 0)
                return 0
            lax.fori_loop(0, H // FB_CHUNK, chunk_body, 0)

        for ch, v in ((0, vr), (1, vg), (2, vb)):
            obase = pl.multiple_of((b * C + ch) * HW + row0 * W, 1024)
            pltpu.sync_copy(v, out_hbm.at[pl.ds(obase, TILE_ELEMS)])


def _sc_warp(x):
    mesh = plsc.VectorSubcoreMesh(core_axis_name="c", subcore_axis_name="s")
    fn = pl.kernel(
        _sc_warp_body,
        mesh=mesh,
        compiler_params=pltpu.CompilerParams(needs_layout_passes=False),
        out_type=jax.ShapeDtypeStruct((B * C * HW,), jnp.float32),
        scratch_types=[
            pltpu.VMEM((WIN, W), jnp.float32),
            pltpu.VMEM((WIN, W), jnp.float32),
            pltpu.VMEM((3, WIN, W), jnp.float32),
            pltpu.VMEM((TILE_ELEMS,), jnp.float32),
            pltpu.VMEM((TILE_ELEMS,), jnp.float32),
            pltpu.VMEM((TILE_ELEMS,), jnp.float32),
            pltpu.VMEM((L,), jnp.int32),
            pltpu.VMEM((NSUB, L), jnp.int32),
            pltpu.VMEM_SHARED((2, NSUB, L), jnp.int32),
        ],
    )
    return fn(x)


def _tc_assemble_body(x_ref, w_ref, o_ref):
    a = x_ref[0]
    w = w_ref[0]
    im2 = a[3:6]
    d = w - im2
    err = jnp.sqrt(d[0] * d[0] + d[1] * d[1] + d[2] * d[2])
    o_ref[0, 0:3] = a[0:3]
    o_ref[0, 3:6] = w
    o_ref[0, 6:9] = im2
    o_ref[0, 9:11] = a[6:8]
    o_ref[0, 11] = err


def _tc_assemble(x, warped):
    rows = 48
    grid = (B, H // rows)
    return pl.pallas_call(
        _tc_assemble_body,
        grid=grid,
        in_specs=[
            pl.BlockSpec((1, 8, rows, W), lambda b, r: (b, 0, r, 0)),
            pl.BlockSpec((1, 3, rows, W), lambda b, r: (b, 0, r, 0)),
        ],
        out_specs=pl.BlockSpec((1, 12, rows, W), lambda b, r: (b, 0, r, 0)),
        out_shape=jax.ShapeDtypeStruct((B, 12, H, W), jnp.float32),
    )(x, warped)


def kernel(x):
    warped = _sc_warp(x).reshape(B, C, H, W)
    return _tc_assemble(x, warped)


# speedup vs baseline: 14.6380x; 4.5336x over previous
import functools

import jax
import jax.numpy as jnp
from jax import lax
from jax.experimental import pallas as pl
from jax.experimental.pallas import tpu as pltpu
from jax.experimental.pallas import tpu_sc as plsc

B, C, H, W = 4, 3, 384, 384
HW = H * W
NCORES, NSUB, L = 2, 16, 16
ROWS_PER_TILE = H // NSUB
TILE_ELEMS = ROWS_PER_TILE * W
HALO = 8
WIN = ROWS_PER_TILE + 2 * HALO
FB_CHUNK = 32
GROUPS = W // L
BIG = 0x7FFFFFFF
MAGIC = 12582912.0


def _rne(x):
    return (x + MAGIC) - MAGIC


def _take16(x, idx):
    return lax.gather(
        x, idx[:, None],
        lax.GatherDimensionNumbers(
            offset_dims=(), collapsed_slice_dims=(0,), start_index_map=(0,)),
        slice_sizes=(1,),
        mode=lax.GatherScatterMode.PROMISE_IN_BOUNDS)


def _sc_warp_body(x_hbm, out_hbm, fxw, fyw, imw, vr, vg, vb, flg_v, flg_all, shared):
    core = lax.axis_index("c")
    sub = lax.axis_index("s")
    row0 = sub * ROWS_PER_TILE
    lane = lax.iota(jnp.int32, L)
    lane_f = lane.astype(jnp.float32)
    nxt = jnp.minimum(lane + 1, L - 1)
    zeros16 = jnp.zeros((L,), jnp.float32)

    def group_core(wr, g, base_row):
        r_glob = base_row + wr
        r_f = r_glob.astype(jnp.float32)
        c0 = g * L
        fx = fxw[wr, pl.ds(c0, L)]
        fy = fyw[wr, pl.ds(c0, L)]
        jj = c0.astype(jnp.float32) + lane_f
        drf = _rne(fy + r_f)
        dcf = _rne(fx + jj)
        vglob = (drf >= 0.0) & (drf < float(H)) & (dcf >= 0.0) & (dcf < float(W))
        dr = jnp.clip(drf, 0.0, float(H - 1)).astype(jnp.int32)
        dc = jnp.clip(dcf, 0.0, float(W - 1)).astype(jnp.int32)
        mine = vglob & (dr >= row0) & (dr < row0 + ROWS_PER_TILE)
        off = (dr - row0) * W + dc
        return r_f, c0, off, mine, vglob, drf

    def scatter_vals(wr, c0, off, keep):
        plsc.store_scatter(vr, [off], imw[0, wr, pl.ds(c0, L)], mask=keep)
        plsc.store_scatter(vg, [off], imw[1, wr, pl.ds(c0, L)], mask=keep)
        plsc.store_scatter(vb, [off], imw[2, wr, pl.ds(c0, L)], mask=keep)

    def group_fast(wr, g, base_row, ofl, bad):
        r_f, c0, off, mine, vglob, drf = group_core(wr, g, base_row)
        plsc.store_scatter(vr, [off], lane_f, mask=mine)
        w = plsc.load_gather(vr, [off], mask=mine)
        keep = mine & (w == lane_f)
        bad = bad | (mine & (w < lane_f)).astype(jnp.int32)
        scatter_vals(wr, c0, off, keep)
        r_glob = base_row + wr
        own = (r_glob >= row0) & (r_glob < row0 + ROWS_PER_TILE)
        far = vglob & (jnp.abs(drf - r_f) > float(HALO))
        ofl = ofl | jnp.where(own, far.astype(jnp.int32), 0)
        return ofl, bad

    def group_precise(wr, g, base_row):
        _, c0, off, mine, _, _ = group_core(wr, g, base_row)
        offu = jnp.where(mine, off, -1 - lane)
        dup_later = jnp.zeros((L,), jnp.bool_)
        for d in range(1, L):
            offd = _take16(offu, jnp.minimum(lane + d, L - 1))
            dup_later = dup_later | ((offd == offu) & (lane + d < L))
        keep = mine & jnp.logical_not(dup_later)
        scatter_vals(wr, c0, off, keep)

    def zero_vals():
        def zbody(i, _):
            vr[pl.ds(i * L, L)] = zeros16
            vg[pl.ds(i * L, L)] = zeros16
            vb[pl.ds(i * L, L)] = zeros16
            return 0
        lax.fori_loop(0, TILE_ELEMS // L, zbody, 0)

    def load_window(b, r_start, nrows):
        r_start = pl.multiple_of(r_start, 8)
        pltpu.sync_copy(x_hbm.at[b, 6, pl.ds(r_start, nrows)], fxw.at[pl.ds(0, nrows)])
        pltpu.sync_copy(x_hbm.at[b, 7, pl.ds(r_start, nrows)], fyw.at[pl.ds(0, nrows)])
        for ch in range(3):
            pltpu.sync_copy(x_hbm.at[b, ch, pl.ds(r_start, nrows)],
                            imw.at[ch, pl.ds(0, nrows)])

    UNROLL = 4

    def precise_rows(base_row, nrows):
        def row_body(wr, _):
            def q_body(q, __):
                for u in range(UNROLL):
                    group_precise(wr, q * UNROLL + u, base_row)
                return 0
            return lax.fori_loop(0, GROUPS // UNROLL, q_body, 0)
        lax.fori_loop(0, nrows, row_body, 0)

    for k in range(2):
        b = core * 2 + k
        zero_vals()

        w0 = jnp.clip(row0 - HALO, 0, H - WIN)
        load_window(b, w0, WIN)

        def row_body(wr, carry):
            def q_body(q, c_in):
                ofl_in, bad_in = c_in
                for u in range(UNROLL):
                    ofl_in, bad_in = group_fast(wr, q * UNROLL + u, w0,
                                                ofl_in, bad_in)
                return ofl_in, bad_in
            return lax.fori_loop(0, GROUPS // UNROLL, q_body, carry)

        z16 = jnp.zeros((L,), jnp.int32)
        ofl, bad = lax.fori_loop(0, WIN, row_body, (z16, z16))

        flg_v[pl.ds(0, L)] = ofl
        pltpu.sync_copy(flg_v, shared.at[k, sub])
        plsc.subcore_barrier()
        pltpu.sync_copy(shared.at[k], flg_all)
        def or_body(i, a):
            return a | flg_all[i, pl.ds(0, L)]
        acc = lax.fori_loop(0, NSUB, or_body, jnp.zeros((L,), jnp.int32))
        any_out = jnp.any(acc != 0)
        any_bad = jnp.any(bad != 0)

        @pl.when(any_bad & jnp.logical_not(any_out))
        def _redo_window():
            zero_vals()
            precise_rows(w0, WIN)

        @pl.when(any_out)
        def _fallback():
            zero_vals()
            def chunk_body(ci, _):
                r_start = ci * FB_CHUNK
                load_window(b, r_start, FB_CHUNK)
                precise_rows(r_start, FB_CHUNK)
                return 0
            lax.fori_loop(0, H // FB_CHUNK, chunk_body, 0)

        for ch, v in ((0, vr), (1, vg), (2, vb)):
            obase = pl.multiple_of((b * C + ch) * HW + row0 * W, 1024)
            pltpu.sync_copy(v, out_hbm.at[pl.ds(obase, TILE_ELEMS)])


def _sc_warp(x):
    mesh = plsc.VectorSubcoreMesh(core_axis_name="c", subcore_axis_name="s")
    fn = pl.kernel(
        _sc_warp_body,
        mesh=mesh,
        compiler_params=pltpu.CompilerParams(needs_layout_passes=False),
        out_type=jax.ShapeDtypeStruct((B * C * HW,), jnp.float32),
        scratch_types=[
            pltpu.VMEM((WIN, W), jnp.float32),
            pltpu.VMEM((WIN, W), jnp.float32),
            pltpu.VMEM((3, WIN, W), jnp.float32),
            pltpu.VMEM((TILE_ELEMS,), jnp.float32),
            pltpu.VMEM((TILE_ELEMS,), jnp.float32),
            pltpu.VMEM((TILE_ELEMS,), jnp.float32),
            pltpu.VMEM((L,), jnp.int32),
            pltpu.VMEM((NSUB, L), jnp.int32),
            pltpu.VMEM_SHARED((2, NSUB, L), jnp.int32),
        ],
    )
    return fn(x)


def _tc_assemble_body(x_ref, w_ref, o_ref):
    a = x_ref[0]
    w = w_ref[0]
    im2 = a[3:6]
    d = w - im2
    err = jnp.sqrt(d[0] * d[0] + d[1] * d[1] + d[2] * d[2])
    o_ref[0, 0:3] = a[0:3]
    o_ref[0, 3:6] = w
    o_ref[0, 6:9] = im2
    o_ref[0, 9:11] = a[6:8]
    o_ref[0, 11] = err


def _tc_assemble(x, warped):
    rows = 48
    grid = (B, H // rows)
    return pl.pallas_call(
        _tc_assemble_body,
        grid=grid,
        in_specs=[
            pl.BlockSpec((1, 8, rows, W), lambda b, r: (b, 0, r, 0)),
            pl.BlockSpec((1, 3, rows, W), lambda b, r: (b, 0, r, 0)),
        ],
        out_specs=pl.BlockSpec((1, 12, rows, W), lambda b, r: (b, 0, r, 0)),
        out_shape=jax.ShapeDtypeStruct((B, 12, H, W), jnp.float32),
    )(x, warped)


def kernel(x):
    warped = _sc_warp(x).reshape(B, C, H, W)
    return _tc_assemble(x, warped)
